# baseline (speedup 1.0000x reference)
import functools

import jax
import jax.numpy as jnp
from jax import lax
from jax.experimental import pallas as pl
from jax.experimental.pallas import tpu as pltpu
from jax.experimental.pallas import tpu_sc as plsc

_D = 768
_T_OUT = 64
_H_OUT = 32
_W_OUT = 32
_NC = 2
_NS = 16
_NW = _NC * _NS
_TPW = _T_OUT // _NW
_C = _D // 16


def _sc_body(scal_hbm, t_hbm, h_hbm, w_hbm, out_hbm,
             scal_v, idx_h, idx_w, h_rows, w_rows, trow, th,
             strip0, strip1, sem_in, sem0, sem1):
    cid = lax.axis_index("c")
    sid = lax.axis_index("s")
    wid = sid * _NC + cid

    pltpu.sync_copy(scal_hbm, scal_v)
    sv = scal_v[pl.ds(0, 16)]
    T = sv[0]
    nh = sv[1]
    nw = sv[2]

    for chunk in range(_H_OUT // 16):
        vec = lax.broadcasted_iota(jnp.int32, (16,), 0) + chunk * 16
        idx_h[pl.ds(chunk * 16, 16)] = jnp.minimum(vec, nh - 1)
        idx_w[pl.ds(chunk * 16, 16)] = jnp.minimum(vec, nw - 1)

    pltpu.async_copy(h_hbm.at[idx_h], h_rows, sem_in).wait()
    pltpu.async_copy(w_hbm.at[idx_w], w_rows, sem_in).wait()

    def strip_into(h, buf):
        def c_body(c, carry):
            thc = th[h, pl.ds(c * 16, 16)]
            for w in range(_W_OUT):
                buf[w, pl.ds(c * 16, 16)] = w_rows[w, pl.ds(c * 16, 16)] + thc
            return carry

        lax.fori_loop(0, _C, c_body, 0)

    for ti in range(_TPW):
        t = wid * _TPW + ti
        t_idx = jnp.minimum(t, T - 1)
        pltpu.sync_copy(t_hbm.at[pl.ds(t_idx, 1)], trow)

        def th_c_body(c, carry):
            tr = trow[0, pl.ds(c * 16, 16)]
            for h in range(_H_OUT):
                th[h, pl.ds(c * 16, 16)] = h_rows[h, pl.ds(c * 16, 16)] + tr
            return carry

        lax.fori_loop(0, _C, th_c_body, 0)

        base_t = t * (_H_OUT * _W_OUT)

        def pair_body(p, carry):
            h0 = p * 2

            @pl.when(p > 0)
            def _wait0():
                pltpu.make_async_copy(
                    strip0, out_hbm.at[pl.ds(base_t, _W_OUT)], sem0).wait()

            strip_into(h0, strip0)
            pltpu.async_copy(
                strip0, out_hbm.at[pl.ds(base_t + h0 * _W_OUT, _W_OUT)],
                sem0).start()

            @pl.when(p > 0)
            def _wait1():
                pltpu.make_async_copy(
                    strip1, out_hbm.at[pl.ds(base_t, _W_OUT)], sem1).wait()

            strip_into(h0 + 1, strip1)
            pltpu.async_copy(
                strip1, out_hbm.at[pl.ds(base_t + (h0 + 1) * _W_OUT, _W_OUT)],
                sem1).start()
            return carry

        lax.fori_loop(0, _H_OUT // 2, pair_body, 0)

        pltpu.make_async_copy(
            strip0, out_hbm.at[pl.ds(base_t, _W_OUT)], sem0).wait()
        pltpu.make_async_copy(
            strip1, out_hbm.at[pl.ds(base_t, _W_OUT)], sem1).wait()


def kernel(T, n_h, n_w, t_w, h_w, w_w):
    scal = jnp.zeros((16,), jnp.int32)
    scal = scal.at[0].set(jnp.asarray(T, jnp.int32))
    scal = scal.at[1].set(jnp.asarray(n_h, jnp.int32))
    scal = scal.at[2].set(jnp.asarray(n_w, jnp.int32))

    mesh = plsc.VectorSubcoreMesh(core_axis_name="c", subcore_axis_name="s")
    run = pl.kernel(
        _sc_body,
        out_type=jax.ShapeDtypeStruct((_T_OUT * _H_OUT * _W_OUT, _D),
                                      jnp.float32),
        mesh=mesh,
        scratch_types=[
            pltpu.VMEM((16,), jnp.int32),
            pltpu.VMEM((_H_OUT,), jnp.int32),
            pltpu.VMEM((_W_OUT,), jnp.int32),
            pltpu.VMEM((_H_OUT, _D), jnp.float32),
            pltpu.VMEM((_W_OUT, _D), jnp.float32),
            pltpu.VMEM((1, _D), jnp.float32),
            pltpu.VMEM((_H_OUT, _D), jnp.float32),
            pltpu.VMEM((_W_OUT, _D), jnp.float32),
            pltpu.VMEM((_W_OUT, _D), jnp.float32),
            pltpu.SemaphoreType.DMA,
            pltpu.SemaphoreType.DMA,
            pltpu.SemaphoreType.DMA,
        ],
    )
    out = run(scal, t_w, h_w, w_w)
    return out.reshape(_T_OUT, _H_OUT, _W_OUT, _D)


# speedup vs baseline: 1.0822x; 1.0822x over previous
import functools

import jax
import jax.numpy as jnp
from jax import lax
from jax.experimental import pallas as pl
from jax.experimental.pallas import tpu as pltpu
from jax.experimental.pallas import tpu_sc as plsc

_D = 768
_T_OUT = 64
_H_OUT = 32
_W_OUT = 32
_NC = 2
_NS = 16
_NW = _NC * _NS
_TPW = _T_OUT // _NW
_C = _D // 16


def _sc_body(scal_hbm, t_hbm, h_hbm, w_hbm, out_hbm,
             scal_v, idx_h, idx_w, h_rows, w_rows, trow, th,
             strip0, strip1, sem_in, sem0, sem1):
    cid = lax.axis_index("c")
    sid = lax.axis_index("s")
    wid = sid * _NC + cid

    pltpu.sync_copy(scal_hbm, scal_v)
    sv = scal_v[pl.ds(0, 16)]
    T = sv[0]
    nh = sv[1]
    nw = sv[2]

    for chunk in range(_H_OUT // 16):
        vec = lax.broadcasted_iota(jnp.int32, (16,), 0) + chunk * 16
        idx_h[pl.ds(chunk * 16, 16)] = jnp.minimum(vec, nh - 1)
        idx_w[pl.ds(chunk * 16, 16)] = jnp.minimum(vec, nw - 1)

    pltpu.async_copy(h_hbm.at[idx_h], h_rows, sem_in).wait()
    pltpu.async_copy(w_hbm.at[idx_w], w_rows, sem_in).wait()

    def strip_into(h, buf):
        thc = [th[h, 16 * c:16 * (c + 1)] for c in range(_C)]

        def w_body(w, carry):
            for c in range(_C):
                buf[w, 16 * c:16 * (c + 1)] = (
                    w_rows[w, 16 * c:16 * (c + 1)] + thc[c])
            return carry

        lax.fori_loop(0, _W_OUT, w_body, 0)

    for ti in range(_TPW):
        t = wid * _TPW + ti
        t_idx = jnp.minimum(t, T - 1)
        pltpu.sync_copy(t_hbm.at[pl.ds(t_idx, 1)], trow)

        def th_c_body(c, carry):
            tr = trow[0, pl.ds(c * 16, 16)]

            def th_h_body(h, inner):
                th[h, pl.ds(c * 16, 16)] = h_rows[h, pl.ds(c * 16, 16)] + tr
                return inner

            return lax.fori_loop(0, _H_OUT, th_h_body, carry)

        lax.fori_loop(0, _C, th_c_body, 0)

        base_t = t * (_H_OUT * _W_OUT)

        def pair_body(p, carry):
            h0 = p * 2
            strip_into(h0, strip0)
            pltpu.sync_copy(
                strip0, out_hbm.at[pl.ds(base_t + h0 * _W_OUT, _W_OUT)])
            strip_into(h0 + 1, strip1)
            pltpu.sync_copy(
                strip1, out_hbm.at[pl.ds(base_t + (h0 + 1) * _W_OUT, _W_OUT)])
            return carry

        lax.fori_loop(0, _H_OUT // 2, pair_body, 0)


def kernel(T, n_h, n_w, t_w, h_w, w_w):
    scal = jnp.zeros((16,), jnp.int32)
    scal = scal.at[0].set(jnp.asarray(T, jnp.int32))
    scal = scal.at[1].set(jnp.asarray(n_h, jnp.int32))
    scal = scal.at[2].set(jnp.asarray(n_w, jnp.int32))

    mesh = plsc.VectorSubcoreMesh(core_axis_name="c", subcore_axis_name="s")
    run = pl.kernel(
        _sc_body,
        out_type=jax.ShapeDtypeStruct((_T_OUT * _H_OUT * _W_OUT, _D),
                                      jnp.float32),
        mesh=mesh,
        scratch_types=[
            pltpu.VMEM((16,), jnp.int32),
            pltpu.VMEM((_H_OUT,), jnp.int32),
            pltpu.VMEM((_W_OUT,), jnp.int32),
            pltpu.VMEM((_H_OUT, _D), jnp.float32),
            pltpu.VMEM((_W_OUT, _D), jnp.float32),
            pltpu.VMEM((1, _D), jnp.float32),
            pltpu.VMEM((_H_OUT, _D), jnp.float32),
            pltpu.VMEM((_W_OUT, _D), jnp.float32),
            pltpu.VMEM((_W_OUT, _D), jnp.float32),
            pltpu.SemaphoreType.DMA,
            pltpu.SemaphoreType.DMA,
            pltpu.SemaphoreType.DMA,
        ],
    )
    out = run(scal, t_w, h_w, w_w)
    return out.reshape(_T_OUT, _H_OUT, _W_OUT, _D)


# speedup vs baseline: 2.6920x; 2.4875x over previous
import jax
import jax.numpy as jnp
from jax import lax
from jax.experimental import pallas as pl
from jax.experimental.pallas import tpu as pltpu

_D = 768
_T_OUT = 64
_H_OUT = 32
_W_OUT = 32
_T_BLK = 2


def _body(scal_ref, t_ref, h_ref, w_ref, out_ref):
    t = pl.program_id(0)
    T = scal_ref[0]
    nh = scal_ref[1]
    nw = scal_ref[2]

    row_ids = lax.broadcasted_iota(jnp.int32, (_H_OUT, 1), 0)
    h_clamp = jnp.maximum(nh - 1, 0)
    h_last = h_ref[pl.ds(h_clamp, 1), :]
    h_rows = jnp.where(row_ids < nh, h_ref[0:_H_OUT, :], h_last)
    w_clamp = jnp.maximum(nw - 1, 0)
    w_last = w_ref[pl.ds(w_clamp, 1), :]
    w_rows = jnp.where(row_ids < nw, w_ref[0:_W_OUT, :], w_last)

    for ti in range(_T_BLK):
        t_idx = jnp.clip(t * _T_BLK + ti, 0, jnp.maximum(T - 1, 0))
        t_row = t_ref[pl.ds(t_idx, 1), :]
        th = h_rows + t_row
        for h in range(_H_OUT):
            out_ref[ti, h] = th[h:h + 1, :] + w_rows


def kernel(T, n_h, n_w, t_w, h_w, w_w):
    scal = jnp.stack([jnp.asarray(T, jnp.int32),
                      jnp.asarray(n_h, jnp.int32),
                      jnp.asarray(n_w, jnp.int32)])
    return pl.pallas_call(
        _body,
        grid=(_T_OUT // _T_BLK,),
        in_specs=[
            pl.BlockSpec(memory_space=pltpu.SMEM),
            pl.BlockSpec((t_w.shape[0], _D), lambda i: (0, 0)),
            pl.BlockSpec((h_w.shape[0], _D), lambda i: (0, 0)),
            pl.BlockSpec((w_w.shape[0], _D), lambda i: (0, 0)),
        ],
        out_specs=pl.BlockSpec((_T_BLK, _H_OUT, _W_OUT, _D),
                               lambda i: (i, 0, 0, 0)),
        out_shape=jax.ShapeDtypeStruct((_T_OUT, _H_OUT, _W_OUT, _D), jnp.float32),
        compiler_params=pltpu.CompilerParams(
            dimension_semantics=("arbitrary",)),
    )(scal, t_w, h_w, w_w)
